# baseline (speedup 1.0000x reference)
import functools

import jax
import jax.numpy as jnp
import numpy as np
from jax.experimental import pallas as pl
from jax.experimental.pallas import tpu as pltpu

_LEVELS = np.array([8, 5, 5, 5])
_G = 4
_NQ = 8
_L = 4
_DIM = 768
_DG = _DIM // _G
_GL = _G * _L
_TB = 2048


def _fsq_body(x_ref, w2t_ref, wout2_ref, bin_ref, bout_ref, isc_ref,
              qm_ref, bc_ref, ones_ref, idx_ref, q_ref, loss_ref, met_ref,
              hist_acc, loss_acc):
    i = pl.program_id(0)
    nsteps = pl.num_programs(0)

    @pl.when(i == 0)
    def _init():
        hist_acc[...] = jnp.zeros_like(hist_acc)
        loss_acc[...] = jnp.zeros_like(loss_acc)

    xblk = x_ref[...]
    z_tok = jnp.concatenate(
        [jax.lax.dot_general(
            xblk[:, g * _DG:(g + 1) * _DG],
            w2t_ref[g * _DG:(g + 1) * _DG, :],
            (((1,), (0,)), ((), ())),
            preferred_element_type=jnp.float32)
         for g in range(_G)], axis=1)
    z = z_tok.T + bin_ref[...]

    half_l = bc_ref[:, 0:1]
    offset = bc_ref[:, 1:2]
    shift = bc_ref[:, 2:3]
    half_w = bc_ref[:, 3:4]

    iota40 = jax.lax.broadcasted_iota(jnp.int32, (1, 40, 1), 1)
    iota25 = jax.lax.broadcasted_iota(jnp.int32, (1, 25, 1), 1)

    resid = z
    qout = jnp.zeros_like(z)
    hists = []
    idxs = []
    for q in range(_NQ):
        inv_scale = isc_ref[:, q:q + 1]
        qmul = qm_ref[:, q:q + 1]
        zq = jnp.tanh(resid * inv_scale + shift) * half_l - offset
        codes = jnp.round(zq)
        quant = codes * qmul
        resid = resid - quant
        qout = qout + quant
        d = (codes + half_w).reshape(_G, _L, _TB)
        p = (d[:, 0, :] + 8.0 * d[:, 1, :]).astype(jnp.int32)
        h = (d[:, 2, :] + 5.0 * d[:, 3, :]).astype(jnp.int32)
        idxs.append(p + 40 * h)
        u = (p[:, None, :] == iota40).astype(jnp.bfloat16)
        v = (h[:, None, :] == iota25).astype(jnp.bfloat16)
        hq = jax.lax.dot_general(
            v, u, (((2,), (2,)), ((0,), (0,))),
            preferred_element_type=jnp.float32)
        hists.append(hq)
    hist_acc[...] += jnp.stack(hists, axis=1)
    idx_ref[...] = jnp.stack(idxs, axis=1).transpose(0, 2, 1)

    out = jax.lax.dot_general(
        qout, wout2_ref[...], (((0,), (0,)), ((), ())),
        preferred_element_type=jnp.float32) + bout_ref[...]
    q_ref[...] = out
    diff = out - xblk
    loss_acc[...] += jax.lax.dot_general(
        ones_ref[...], diff * diff, (((1,), (0,)), ((), ())),
        preferred_element_type=jnp.float32)

    @pl.when(i == nsteps - 1)
    def _fin():
        ntok = nsteps * _TB
        loss_ref[...] = jnp.sum(loss_acc[...], axis=1, keepdims=True) * (
            1.0 / float(ntok * _DIM))
        probs = hist_acc[...] * (1.0 / float(ntok))
        plogp = jnp.where(probs > 0, probs * jnp.log(probs + 1e-10), 0.0)
        ent = -jnp.sum(jnp.sum(plogp, axis=3), axis=2)
        met_ref[...] = jnp.exp(ent)


@jax.jit
def kernel(x, w_in, b_in, w_out, b_out):
    B, T, D = x.shape
    ntok = B * T
    nsteps = ntok // _TB
    xf = x.reshape(ntok, D)

    w2t = w_in.reshape(_G * _DG, _L)
    wout2 = jax.scipy.linalg.block_diag(*[w_out[g] for g in range(_G)])
    bin_c = b_in.reshape(_GL, 1)
    bout_r = b_out.reshape(1, D)

    levels = jnp.tile(jnp.asarray(_LEVELS, jnp.float32), _G)
    eps = 1e-3
    half_l = (levels - 1.0) * (1.0 - eps) / 2.0
    offset = jnp.tile(jnp.where(jnp.asarray(_LEVELS % 2 == 0), 0.5, 0.0), _G)
    shift = jnp.arctanh(offset / half_l)
    half_w = jnp.tile(jnp.asarray(_LEVELS // 2, jnp.float32), _G)
    qs = jnp.arange(_NQ, dtype=jnp.float32)
    inv_scales = (levels - 1.0)[:, None] ** qs[None, :]
    scales = (levels - 1.0)[:, None] ** (-qs[None, :])
    qmuls = scales / half_w[:, None]
    bconsts = jnp.stack([half_l, offset, shift, half_w], axis=1)
    ones_row = jnp.ones((1, _TB), jnp.float32)

    const_spec = pl.BlockSpec(index_map=lambda i: (0, 0))
    idx_t, qf, loss, met = pl.pallas_call(
        _fsq_body,
        grid=(nsteps,),
        in_specs=[
            pl.BlockSpec((_TB, D), lambda i: (i, 0)),
            const_spec, const_spec, const_spec, const_spec, const_spec,
            const_spec, const_spec, const_spec,
        ],
        out_specs=[
            pl.BlockSpec((_G, _TB, _NQ), lambda i: (0, i, 0)),
            pl.BlockSpec((_TB, D), lambda i: (i, 0)),
            pl.BlockSpec((1, 1), lambda i: (0, 0)),
            pl.BlockSpec((_G, _NQ), lambda i: (0, 0)),
        ],
        out_shape=[
            jax.ShapeDtypeStruct((_G, ntok, _NQ), jnp.int32),
            jax.ShapeDtypeStruct((ntok, D), jnp.float32),
            jax.ShapeDtypeStruct((1, 1), jnp.float32),
            jax.ShapeDtypeStruct((_G, _NQ), jnp.float32),
        ],
        scratch_shapes=[
            pltpu.VMEM((_G, _NQ, 25, 40), jnp.float32),
            pltpu.VMEM((1, _DIM), jnp.float32),
        ],
        compiler_params=pltpu.CompilerParams(
            dimension_semantics=("arbitrary",)),
    )(xf, w2t, wout2, bin_c, bout_r, inv_scales, qmuls, bconsts, ones_row)

    all_indices = idx_t.reshape(_G, B, T, _NQ)
    quantized = qf.reshape(B, T, D)
    return (all_indices, quantized, loss.reshape(()), met)


# speedup vs baseline: 1.3806x; 1.3806x over previous
import functools

import jax
import jax.numpy as jnp
import numpy as np
from jax.experimental import pallas as pl
from jax.experimental.pallas import tpu as pltpu

_LEVELS = np.array([8, 5, 5, 5])
_G = 4
_NQ = 8
_L = 4
_DIM = 768
_DG = _DIM // _G
_GL = _G * _L
_TB = 2048


def _fsq_body(x_ref, w2t_ref, wout2_ref, bin_ref, bout_ref, isc_ref,
              qm_ref, bc_ref, ones_ref, idx_ref, q_ref, loss_ref, met_ref,
              hist_acc, loss_acc):
    i = pl.program_id(0)
    nsteps = pl.num_programs(0)

    @pl.when(i == 0)
    def _init():
        hist_acc[...] = jnp.zeros_like(hist_acc)
        loss_acc[...] = jnp.zeros_like(loss_acc)

    xblk = x_ref[...]
    z_tok = jnp.concatenate(
        [jax.lax.dot_general(
            xblk[:, g * _DG:(g + 1) * _DG],
            w2t_ref[g * _DG:(g + 1) * _DG, :],
            (((1,), (0,)), ((), ())),
            preferred_element_type=jnp.float32)
         for g in range(_G)], axis=1)
    z = z_tok.T + bin_ref[...]

    half_l = bc_ref[:, 0:1]
    offset = bc_ref[:, 1:2]
    shift = bc_ref[:, 2:3]
    half_w = bc_ref[:, 3:4]

    iota40 = jax.lax.broadcasted_iota(jnp.int32, (1, 40, 1), 1)
    iota25 = jax.lax.broadcasted_iota(jnp.int32, (1, 25, 1), 1)

    resid = z
    qout = jnp.zeros_like(z)
    hists = []
    idxs = []
    for q in range(_NQ):
        inv_scale = isc_ref[:, q:q + 1]
        qmul = qm_ref[:, q:q + 1]
        zq = jnp.tanh(resid * inv_scale + shift) * half_l - offset
        codes = jnp.round(zq)
        quant = codes * qmul
        resid = resid - quant
        qout = qout + quant
        d = (codes + half_w).reshape(_G, _L, _TB)
        p = (d[:, 0, :] + 8.0 * d[:, 1, :]).astype(jnp.int32)
        h = (d[:, 2, :] + 5.0 * d[:, 3, :]).astype(jnp.int32)
        idxs.append(p + 40 * h)
        u = (p[:, None, :] == iota40).astype(jnp.bfloat16)
        v = (h[:, None, :] == iota25).astype(jnp.bfloat16)
        hq = jax.lax.dot_general(
            v, u, (((2,), (2,)), ((0,), (0,))),
            preferred_element_type=jnp.float32)
        hists.append(hq)
    hist_acc[...] += jnp.stack(hists, axis=1)
    idx_ref[...] = jnp.stack(idxs, axis=1)

    out = jax.lax.dot_general(
        qout, wout2_ref[...], (((0,), (0,)), ((), ())),
        preferred_element_type=jnp.float32) + bout_ref[...]
    q_ref[...] = out
    diff = out - xblk
    loss_acc[...] += jax.lax.dot_general(
        ones_ref[...], diff * diff, (((1,), (0,)), ((), ())),
        preferred_element_type=jnp.float32)

    @pl.when(i == nsteps - 1)
    def _fin():
        ntok = nsteps * _TB
        loss_ref[...] = jnp.sum(loss_acc[...], axis=1, keepdims=True) * (
            1.0 / float(ntok * _DIM))
        probs = hist_acc[...] * (1.0 / float(ntok))
        plogp = jnp.where(probs > 0, probs * jnp.log(probs + 1e-10), 0.0)
        ent = -jnp.sum(jnp.sum(plogp, axis=3), axis=2)
        met_ref[...] = jnp.exp(ent)


@jax.jit
def kernel(x, w_in, b_in, w_out, b_out):
    B, T, D = x.shape
    ntok = B * T
    nsteps = ntok // _TB
    xf = x.reshape(ntok, D)

    w2t = w_in.reshape(_G * _DG, _L)
    wout2 = jax.scipy.linalg.block_diag(*[w_out[g] for g in range(_G)])
    bin_c = b_in.reshape(_GL, 1)
    bout_r = b_out.reshape(1, D)

    levels = jnp.tile(jnp.asarray(_LEVELS, jnp.float32), _G)
    eps = 1e-3
    half_l = (levels - 1.0) * (1.0 - eps) / 2.0
    offset = jnp.tile(jnp.where(jnp.asarray(_LEVELS % 2 == 0), 0.5, 0.0), _G)
    shift = jnp.arctanh(offset / half_l)
    half_w = jnp.tile(jnp.asarray(_LEVELS // 2, jnp.float32), _G)
    qs = jnp.arange(_NQ, dtype=jnp.float32)
    inv_scales = (levels - 1.0)[:, None] ** qs[None, :]
    scales = (levels - 1.0)[:, None] ** (-qs[None, :])
    qmuls = scales / half_w[:, None]
    bconsts = jnp.stack([half_l, offset, shift, half_w], axis=1)
    ones_row = jnp.ones((1, _TB), jnp.float32)

    const_spec = pl.BlockSpec(index_map=lambda i: (0, 0))
    idx_t, qf, loss, met = pl.pallas_call(
        _fsq_body,
        grid=(nsteps,),
        in_specs=[
            pl.BlockSpec((_TB, D), lambda i: (i, 0)),
            const_spec, const_spec, const_spec, const_spec, const_spec,
            const_spec, const_spec, const_spec,
        ],
        out_specs=[
            pl.BlockSpec((_G, _NQ, _TB), lambda i: (0, 0, i)),
            pl.BlockSpec((_TB, D), lambda i: (i, 0)),
            pl.BlockSpec((1, 1), lambda i: (0, 0)),
            pl.BlockSpec((_G, _NQ), lambda i: (0, 0)),
        ],
        out_shape=[
            jax.ShapeDtypeStruct((_G, _NQ, ntok), jnp.int32),
            jax.ShapeDtypeStruct((ntok, D), jnp.float32),
            jax.ShapeDtypeStruct((1, 1), jnp.float32),
            jax.ShapeDtypeStruct((_G, _NQ), jnp.float32),
        ],
        scratch_shapes=[
            pltpu.VMEM((_G, _NQ, 25, 40), jnp.float32),
            pltpu.VMEM((1, _DIM), jnp.float32),
        ],
        compiler_params=pltpu.CompilerParams(
            dimension_semantics=("arbitrary",)),
    )(xf, w2t, wout2, bin_c, bout_r, inv_scales, qmuls, bconsts, ones_row)

    all_indices = idx_t.transpose(0, 2, 1).reshape(_G, B, T, _NQ)
    quantized = qf.reshape(B, T, D)
    return (all_indices, quantized, loss.reshape(()), met)


# speedup vs baseline: 1.3814x; 1.0006x over previous
import jax
import jax.numpy as jnp
import numpy as np
from jax.experimental import pallas as pl
from jax.experimental.pallas import tpu as pltpu

_LEVELS = np.array([8, 5, 5, 5])
_G = 4
_NQ = 8
_L = 4
_DIM = 768
_DG = _DIM // _G
_GL = _G * _L
_TB = 2048


def _fsq_body(x_ref, w2t_ref, wout2_ref, bin_ref, bout_ref, isc_ref,
              qm_ref, bc_ref, ones_ref, idx_ref, q_ref, loss_ref, met_ref,
              hist_acc, loss_acc):
    i = pl.program_id(0)
    nsteps = pl.num_programs(0)

    @pl.when(i == 0)
    def _init():
        hist_acc[...] = jnp.zeros_like(hist_acc)
        loss_acc[...] = jnp.zeros_like(loss_acc)

    xblk = x_ref[...]
    z_tok = jnp.concatenate(
        [jax.lax.dot_general(
            xblk[:, g * _DG:(g + 1) * _DG],
            w2t_ref[g * _DG:(g + 1) * _DG, :],
            (((1,), (0,)), ((), ())),
            preferred_element_type=jnp.float32)
         for g in range(_G)], axis=1)
    z = z_tok.T + bin_ref[...]

    half_l = bc_ref[:, 0:1]
    offset = bc_ref[:, 1:2]
    shift = bc_ref[:, 2:3]
    half_w = bc_ref[:, 3:4]

    iota40 = jax.lax.broadcasted_iota(jnp.int32, (1, 40, 1), 1)
    iota25 = jax.lax.broadcasted_iota(jnp.int32, (1, 25, 1), 1)

    resid = z
    qout = jnp.zeros_like(z)
    hists = []
    idxs = []
    for q in range(_NQ):
        inv_scale = isc_ref[:, q:q + 1]
        qmul = qm_ref[:, q:q + 1]
        zq = jnp.tanh(resid * inv_scale + shift) * half_l - offset
        codes = jnp.round(zq)
        quant = codes * qmul
        resid = resid - quant
        qout = qout + quant
        d = (codes + half_w).reshape(_G, _L, _TB)
        p = (d[:, 0, :] + 8.0 * d[:, 1, :]).astype(jnp.int32)
        h = (d[:, 2, :] + 5.0 * d[:, 3, :]).astype(jnp.int32)
        idxs.append(p + 40 * h)
        u = (p[:, None, :] == iota40).astype(jnp.bfloat16)
        v = (h[:, None, :] == iota25).astype(jnp.bfloat16)
        hq = jax.lax.dot_general(
            v, u, (((2,), (2,)), ((0,), (0,))),
            preferred_element_type=jnp.float32)
        hists.append(hq)
    hist_acc[...] += jnp.stack(hists, axis=1)
    idx_ref[...] = jnp.stack(idxs, axis=1)

    out = jax.lax.dot_general(
        qout, wout2_ref[...], (((0,), (0,)), ((), ())),
        preferred_element_type=jnp.float32) + bout_ref[...]
    q_ref[...] = out
    diff = out - xblk
    loss_acc[...] += jax.lax.dot_general(
        ones_ref[...], diff * diff, (((1,), (0,)), ((), ())),
        preferred_element_type=jnp.float32)

    @pl.when(i == nsteps - 1)
    def _fin():
        ntok = nsteps * _TB
        loss_ref[...] = jnp.sum(loss_acc[...], axis=1, keepdims=True) * (
            1.0 / float(ntok * _DIM))
        probs = hist_acc[...] * (1.0 / float(ntok))
        plogp = jnp.where(probs > 0, probs * jnp.log(probs + 1e-10), 0.0)
        ent = -jnp.sum(jnp.sum(plogp, axis=3), axis=2)
        met_ref[...] = jnp.exp(ent)


@jax.jit
def kernel(x, w_in, b_in, w_out, b_out):
    B, T, D = x.shape
    ntok = B * T
    nsteps = ntok // _TB
    xf = x.reshape(ntok, D)

    w2t = w_in.reshape(_G * _DG, _L)
    wout2 = jax.scipy.linalg.block_diag(*[w_out[g] for g in range(_G)])
    bin_c = b_in.reshape(_GL, 1)
    bout_r = b_out.reshape(1, D)

    levels = jnp.tile(jnp.asarray(_LEVELS, jnp.float32), _G)
    eps = 1e-3
    half_l = (levels - 1.0) * (1.0 - eps) / 2.0
    offset = jnp.tile(jnp.where(jnp.asarray(_LEVELS % 2 == 0), 0.5, 0.0), _G)
    shift = jnp.arctanh(offset / half_l)
    half_w = jnp.tile(jnp.asarray(_LEVELS // 2, jnp.float32), _G)
    qs = jnp.arange(_NQ, dtype=jnp.float32)
    inv_scales = (levels - 1.0)[:, None] ** qs[None, :]
    scales = (levels - 1.0)[:, None] ** (-qs[None, :])
    qmuls = scales / half_w[:, None]
    bconsts = jnp.stack([half_l, offset, shift, half_w], axis=1)
    ones_row = jnp.ones((1, _TB), jnp.float32)

    const_spec = pl.BlockSpec(index_map=lambda i: (0, 0))
    idx_t, qf, loss, met = pl.pallas_call(
        _fsq_body,
        grid=(nsteps,),
        in_specs=[
            pl.BlockSpec((_TB, D), lambda i: (i, 0)),
            const_spec, const_spec, const_spec, const_spec, const_spec,
            const_spec, const_spec, const_spec,
        ],
        out_specs=[
            pl.BlockSpec((_G, _NQ, _TB), lambda i: (0, 0, i)),
            pl.BlockSpec((_TB, D), lambda i: (i, 0)),
            pl.BlockSpec((1, 1), lambda i: (0, 0)),
            pl.BlockSpec((_G, _NQ), lambda i: (0, 0)),
        ],
        out_shape=[
            jax.ShapeDtypeStruct((_G, _NQ, ntok), jnp.int32),
            jax.ShapeDtypeStruct((ntok, D), jnp.float32),
            jax.ShapeDtypeStruct((1, 1), jnp.float32),
            jax.ShapeDtypeStruct((_G, _NQ), jnp.float32),
        ],
        scratch_shapes=[
            pltpu.VMEM((_G, _NQ, 25, 40), jnp.float32),
            pltpu.VMEM((1, _DIM), jnp.float32),
        ],
        compiler_params=pltpu.CompilerParams(
            dimension_semantics=("arbitrary",)),
    )(xf, w2t, wout2, bin_c, bout_r, inv_scales, qmuls, bconsts, ones_row)

    all_indices = idx_t.transpose(0, 2, 1).reshape(_G, B, T, _NQ)
    quantized = qf.reshape(B, T, D)
    return (all_indices, quantized, loss.reshape(()), met)
